# baseline (speedup 1.0000x reference)
import functools

import jax
import jax.numpy as jnp
from jax import lax
from jax.experimental import pallas as pl
from jax.experimental.pallas import tpu as pltpu
from jax.experimental.pallas import tpu_sc as plsc

_BATCH = 16384
_DIM = 64
_QROWS = 500000

_INFO = plsc.get_sparse_core_info()
_NC = _INFO.num_cores
_NS = _INFO.num_subcores
_NW = _NC * _NS
_BPW = _BATCH // _NW
_CH = 128
_NCH = _BPW // _CH

_mesh = plsc.VectorSubcoreMesh(core_axis_name="c", subcore_axis_name="s")


@functools.partial(
    pl.kernel,
    mesh=_mesh,
    out_type=jax.ShapeDtypeStruct((_BATCH // 2, 2 * _DIM), jnp.float32),
    scratch_types=[
        pltpu.VMEM((_NCH, _CH), jnp.int32),
        pltpu.VMEM((_NCH, _CH), jnp.int32),
        pltpu.VMEM((_NCH, _CH, 2 * _DIM), jnp.float32),
        pltpu.VMEM((_BPW // 2, 2 * _DIM), jnp.float32),
        pltpu.SemaphoreType.DMA,
    ],
    compiler_params=pltpu.CompilerParams(
        use_tc_tiling_on_sc=True, needs_layout_passes=False
    ),
)
def _sc_gather(q_hbm, p_hbm, tbl_hbm, out_hbm, q_v, p_v, rows_v, out_v, sem):
    wid = lax.axis_index("s") * _NC + lax.axis_index("c")
    pltpu.sync_copy(q_hbm.at[wid], q_v)
    pltpu.sync_copy(p_hbm.at[wid], p_v)
    copies = [
        pltpu.async_copy(tbl_hbm.at[q_v.at[j]], rows_v.at[j], sem)
        for j in range(_NCH)
    ]
    for c in copies:
        c.wait()

    lanes = lax.iota(jnp.int32, 16)

    def body(i, carry):
        j = i >> 7
        r = i & 127
        jv = jnp.full((16,), j, jnp.int32)
        rv = jnp.full((16,), r, jnp.int32)
        pv = plsc.load_gather(p_v, [jv, rv])
        base = pv * _DIM
        u = i >> 1
        h = (i & 1) * _DIM
        for k in range(_DIM // 16):
            colv = base + k * 16 + lanes
            vals = plsc.load_gather(rows_v, [jv, rv, colv])
            out_v[u, pl.ds(h + k * 16, 16)] = vals
        return carry

    lax.fori_loop(0, _BPW, body, 0)
    pltpu.sync_copy(out_v, out_hbm.at[pl.ds(wid * (_BPW // 2), _BPW // 2)])


def kernel(speaker_id, embedding_weight):
    sid = speaker_id.astype(jnp.int32)
    q = (sid >> 1).reshape(_NW, _NCH, _CH)
    p = (sid & 1).reshape(_NW, _NCH, _CH)
    tbl = embedding_weight.reshape(_QROWS, 2 * _DIM)
    out2 = _sc_gather(q, p, tbl)
    return out2.reshape(_BATCH, _DIM)
